# baseline (speedup 1.0000x reference)
import functools

import numpy as np
import jax
import jax.numpy as jnp
from jax import lax
from jax.experimental import pallas as pl
from jax.experimental.pallas import tpu as pltpu
from jax.experimental.pallas import tpu_sc as plsc

_N_FEAT = 26
_N_DIM = 64
_BATCH = 1024
_N_PAIR = (_N_FEAT * (_N_FEAT + 1)) // 2
_OUT_D = 3 * _N_DIM + 2
_LANES = 128
_N_Q = 4
_Q = 88


def _build_pair_tables():
    xi, yi = [], []
    for i in range(_N_FEAT):
        for j in range(i, _N_FEAT):
            xi.append(i * _N_FEAT + j)
            yi.append(j * _N_FEAT + i)
    xi += [0] * 17
    yi += [0] * 17
    return np.asarray(xi, np.int32), np.asarray(yi, np.int32)


_XF, _YF = _build_pair_tables()


def _ffmp_sc_body(inp_hbm, xf_hbm, yf_hbm, out_hbm,
                  xf_v, yf_v,
                  xb0, yb0, ob0, xb1, yb1, ob1,
                  sx0, sy0, so0, sx1, sy1, so1, sp0, sp1):
    wid = lax.axis_index("s") * 2 + lax.axis_index("c")
    q = wid // 8
    b0 = (wid % 8) * _LANES
    k_lo = q * _Q
    k_hi = jnp.minimum(k_lo + _Q, _N_PAIR)
    k_last = k_hi - 1

    pltpu.sync_copy(xf_hbm, xf_v)
    pltpu.sync_copy(yf_hbm, yf_v)

    nd16 = _LANES // 16
    sets = ((xb0, yb0, ob0, sx0, sy0, so0, sp0),
            (xb1, yb1, ob1, sx1, sy1, so1, sp1))

    def clamp(k):
        return jnp.minimum(k, k_last)

    def start_in(k, st):
        xb, yb, _, sx, sy, _, _ = st
        xf = xf_v[0, pl.ds(k, 16)][0]
        yf = yf_v[0, pl.ds(k, 16)][0]
        pltpu.async_copy(inp_hbm.at[xf, :, pl.ds(b0, _LANES)], xb, sx)
        pltpu.async_copy(inp_hbm.at[yf, :, pl.ds(b0, _LANES)], yb, sy)

    def wait_in(st):
        xb, yb, _, sx, sy, _, _ = st
        pltpu.make_async_copy(inp_hbm.at[0, :, pl.ds(b0, _LANES)], xb, sx).wait()
        pltpu.make_async_copy(inp_hbm.at[0, :, pl.ds(b0, _LANES)], yb, sy).wait()

    def start_out(k, st):
        _, _, ob, _, _, so, sp = st
        h = _OUT_D // 2
        pltpu.async_copy(
            ob.at[pl.ds(0, h)], out_hbm.at[pl.ds(0, h), k, pl.ds(b0, _LANES)], so)
        pltpu.async_copy(
            ob.at[pl.ds(h, _OUT_D - h)],
            out_hbm.at[pl.ds(h, _OUT_D - h), k, pl.ds(b0, _LANES)], sp)

    def wait_out(st):
        _, _, ob, _, _, so, sp = st
        h = _OUT_D // 2
        pltpu.make_async_copy(
            ob.at[pl.ds(0, h)], out_hbm.at[pl.ds(0, h), 0, pl.ds(b0, _LANES)], so
        ).wait()
        pltpu.make_async_copy(
            ob.at[pl.ds(h, _OUT_D - h)],
            out_hbm.at[pl.ds(h, _OUT_D - h), 0, pl.ds(b0, _LANES)], sp
        ).wait()

    def compute(st):
        xb, yb, ob = st[0], st[1], st[2]

        def dim_body(d, accs):
            new = []
            for lc in range(nd16):
                xv = xb[d, pl.ds(lc * 16, 16)]
                yv = yb[d, pl.ds(lc * 16, 16)]
                s = xv + yv
                dd = xv - yv
                p = xv * yv
                ob[d, pl.ds(lc * 16, 16)] = s
                ob[_N_DIM + d, pl.ds(lc * 16, 16)] = dd
                ob[2 * _N_DIM + d, pl.ds(lc * 16, 16)] = p
                new.append(accs[2 * lc] + p)
                new.append(accs[2 * lc + 1] + dd * dd)
            return tuple(new)

        zeros = tuple(jnp.zeros((16,), jnp.float32) for _ in range(2 * nd16))
        accs = lax.fori_loop(0, _N_DIM, dim_body, zeros)

        for lc in range(nd16):
            ipa = accs[2 * lc]
            sqa = accs[2 * lc + 1]
            ob[3 * _N_DIM, pl.ds(lc * 16, 16)] = ipa
            bits = lax.bitcast_convert_type(sqa, jnp.int32)
            y0 = lax.bitcast_convert_type(
                jnp.int32(0x5F3759DF) - lax.shift_right_logical(bits, 1),
                jnp.float32)
            for _ in range(3):
                y0 = y0 * (1.5 - 0.5 * sqa * y0 * y0)
            dist = jnp.where(sqa > 0.0, sqa * y0, 0.0)
            ob[3 * _N_DIM + 1, pl.ds(lc * 16, 16)] = dist

    start_in(clamp(k_lo), sets[0])
    start_in(clamp(k_lo + 1), sets[1])

    def step(s, carry):
        for half in range(2):
            st = sets[half]
            k = clamp(k_lo + 2 * s + half)
            wait_in(st)

            @pl.when(s >= 1)
            def _():
                wait_out(st)

            compute(st)
            start_in(clamp(k_lo + 2 * s + half + 2), st)
            start_out(k, st)
        return carry

    lax.fori_loop(0, _Q // 2, step, 0)

    for st in sets:
        wait_in(st)
        wait_out(st)


@functools.cache
def _ffmp_sc():
    mesh = plsc.VectorSubcoreMesh(
        core_axis_name="c", subcore_axis_name="s", num_cores=2, num_subcores=16)
    return pl.kernel(
        _ffmp_sc_body,
        out_type=jax.ShapeDtypeStruct((_OUT_D, _N_PAIR, _BATCH), jnp.float32),
        mesh=mesh,
        compiler_params=pltpu.CompilerParams(needs_layout_passes=False),
        scratch_types=[
            pltpu.VMEM((1, _N_PAIR + 17), jnp.int32),
            pltpu.VMEM((1, _N_PAIR + 17), jnp.int32),
            pltpu.VMEM((_N_DIM, _LANES), jnp.float32),
            pltpu.VMEM((_N_DIM, _LANES), jnp.float32),
            pltpu.VMEM((_OUT_D, _LANES), jnp.float32),
            pltpu.VMEM((_N_DIM, _LANES), jnp.float32),
            pltpu.VMEM((_N_DIM, _LANES), jnp.float32),
            pltpu.VMEM((_OUT_D, _LANES), jnp.float32),
            pltpu.SemaphoreType.DMA,
            pltpu.SemaphoreType.DMA,
            pltpu.SemaphoreType.DMA,
            pltpu.SemaphoreType.DMA,
            pltpu.SemaphoreType.DMA,
            pltpu.SemaphoreType.DMA,
            pltpu.SemaphoreType.DMA,
            pltpu.SemaphoreType.DMA,
        ],
    )


def kernel(input):
    inp_t = jnp.transpose(input, (1, 2, 0))
    out_t = _ffmp_sc()(
        inp_t,
        jnp.asarray(_XF).reshape(1, _N_PAIR + 17),
        jnp.asarray(_YF).reshape(1, _N_PAIR + 17),
    )
    return jnp.transpose(out_t, (2, 1, 0))


# speedup vs baseline: 1.0048x; 1.0048x over previous
import functools

import numpy as np
import jax
import jax.numpy as jnp
from jax import lax
from jax.experimental import pallas as pl
from jax.experimental.pallas import tpu as pltpu
from jax.experimental.pallas import tpu_sc as plsc

_N_FEAT = 26
_N_DIM = 64
_BATCH = 1024
_N_PAIR = (_N_FEAT * (_N_FEAT + 1)) // 2
_OUT_D = 3 * _N_DIM + 2
_LANES = 128
_N_Q = 4
_Q = 88


def _build_pair_tables():
    xi, yi = [], []
    for i in range(_N_FEAT):
        for j in range(i, _N_FEAT):
            xi.append(i * _N_FEAT + j)
            yi.append(j * _N_FEAT + i)
    xi += [0] * 17
    yi += [0] * 17
    return np.asarray(xi, np.int32), np.asarray(yi, np.int32)


_XF, _YF = _build_pair_tables()


def _ffmp_sc_body(inp_hbm, xf_hbm, yf_hbm, out_hbm,
                  xf_v, yf_v,
                  xb0, yb0, ob0, xb1, yb1, ob1,
                  sx0, sy0, so0, sx1, sy1, so1):
    wid = lax.axis_index("s") * 2 + lax.axis_index("c")
    q = wid // 8
    b0 = (wid % 8) * _LANES
    k_lo = q * _Q
    k_hi = jnp.minimum(k_lo + _Q, _N_PAIR)
    k_last = k_hi - 1

    pltpu.sync_copy(xf_hbm, xf_v)
    pltpu.sync_copy(yf_hbm, yf_v)

    nd16 = _LANES // 16
    sets = ((xb0, yb0, ob0, sx0, sy0, so0), (xb1, yb1, ob1, sx1, sy1, so1))

    def clamp(k):
        return jnp.minimum(k, k_last)

    def start_in(k, st):
        xb, yb, _, sx, sy, _ = st
        xf = xf_v[0, pl.ds(k, 16)][0]
        yf = yf_v[0, pl.ds(k, 16)][0]
        pltpu.async_copy(inp_hbm.at[xf, :, pl.ds(b0, _LANES)], xb, sx)
        pltpu.async_copy(inp_hbm.at[yf, :, pl.ds(b0, _LANES)], yb, sy)

    def wait_in(st):
        xb, yb, _, sx, sy, _ = st
        pltpu.make_async_copy(inp_hbm.at[0, :, pl.ds(b0, _LANES)], xb, sx).wait()
        pltpu.make_async_copy(inp_hbm.at[0, :, pl.ds(b0, _LANES)], yb, sy).wait()

    def start_out(k, st):
        _, _, ob, _, _, so = st
        pltpu.async_copy(ob, out_hbm.at[:, k, pl.ds(b0, _LANES)], so)

    def wait_out(st):
        _, _, ob, _, _, so = st
        pltpu.make_async_copy(ob, out_hbm.at[:, 0, pl.ds(b0, _LANES)], so).wait()

    def compute(st):
        xb, yb, ob, _, _, _ = st

        def dim_body(d, accs):
            new = []
            for lc in range(nd16):
                xv = xb[d, pl.ds(lc * 16, 16)]
                yv = yb[d, pl.ds(lc * 16, 16)]
                s = xv + yv
                dd = xv - yv
                p = xv * yv
                ob[d, pl.ds(lc * 16, 16)] = s
                ob[_N_DIM + d, pl.ds(lc * 16, 16)] = dd
                ob[2 * _N_DIM + d, pl.ds(lc * 16, 16)] = p
                new.append(accs[2 * lc] + p)
                new.append(accs[2 * lc + 1] + dd * dd)
            return tuple(new)

        zeros = tuple(jnp.zeros((16,), jnp.float32) for _ in range(2 * nd16))
        accs = lax.fori_loop(0, _N_DIM, dim_body, zeros)

        for lc in range(nd16):
            ipa = accs[2 * lc]
            sqa = accs[2 * lc + 1]
            ob[3 * _N_DIM, pl.ds(lc * 16, 16)] = ipa
            bits = lax.bitcast_convert_type(sqa, jnp.int32)
            y0 = lax.bitcast_convert_type(
                jnp.int32(0x5F3759DF) - lax.shift_right_logical(bits, 1),
                jnp.float32)
            for _ in range(3):
                y0 = y0 * (1.5 - 0.5 * sqa * y0 * y0)
            dist = jnp.where(sqa > 0.0, sqa * y0, 0.0)
            ob[3 * _N_DIM + 1, pl.ds(lc * 16, 16)] = dist

    start_in(clamp(k_lo), sets[0])
    start_in(clamp(k_lo + 1), sets[1])

    def step(s, carry):
        for half in range(2):
            st = sets[half]
            k = clamp(k_lo + 2 * s + half)
            wait_in(st)

            @pl.when(s >= 1)
            def _():
                wait_out(st)

            compute(st)
            start_in(clamp(k_lo + 2 * s + half + 2), st)
            start_out(k, st)
        return carry

    lax.fori_loop(0, _Q // 2, step, 0)

    for st in sets:
        wait_in(st)
        wait_out(st)


@functools.cache
def _ffmp_sc():
    mesh = plsc.VectorSubcoreMesh(
        core_axis_name="c", subcore_axis_name="s", num_cores=2, num_subcores=16)
    return pl.kernel(
        _ffmp_sc_body,
        out_type=jax.ShapeDtypeStruct((_OUT_D, _N_PAIR, _BATCH), jnp.float32),
        mesh=mesh,
        compiler_params=pltpu.CompilerParams(needs_layout_passes=False),
        scratch_types=[
            pltpu.VMEM((1, _N_PAIR + 17), jnp.int32),
            pltpu.VMEM((1, _N_PAIR + 17), jnp.int32),
            pltpu.VMEM((_N_DIM, _LANES), jnp.float32),
            pltpu.VMEM((_N_DIM, _LANES), jnp.float32),
            pltpu.VMEM((_OUT_D, _LANES), jnp.float32),
            pltpu.VMEM((_N_DIM, _LANES), jnp.float32),
            pltpu.VMEM((_N_DIM, _LANES), jnp.float32),
            pltpu.VMEM((_OUT_D, _LANES), jnp.float32),
            pltpu.SemaphoreType.DMA,
            pltpu.SemaphoreType.DMA,
            pltpu.SemaphoreType.DMA,
            pltpu.SemaphoreType.DMA,
            pltpu.SemaphoreType.DMA,
            pltpu.SemaphoreType.DMA,
        ],
    )


def kernel(input):
    inp_t = jnp.transpose(input, (1, 2, 0))
    out_t = _ffmp_sc()(
        inp_t,
        jnp.asarray(_XF).reshape(1, _N_PAIR + 17),
        jnp.asarray(_YF).reshape(1, _N_PAIR + 17),
    )
    return jnp.transpose(out_t, (2, 1, 0))
